# baseline (speedup 1.0000x reference)
import functools

import jax
import jax.numpy as jnp
from jax import lax
from jax.experimental import pallas as pl
from jax.experimental.pallas import tpu as pltpu
from jax.experimental.pallas import tpu_sc as plsc

_B, _Q, _C = 64, 2048, 256
_R = _B * _Q
_NC, _NS = 2, 16
_NW = _NC * _NS
_RW = _R // _NW
_CH = 128
_NCHUNK = _RW // _CH
_GROUPS = _CH // 16
_UNROLL = 4


def _sc_body(lg_hbm, bx_hbm, lab_hbm, bout_hbm, cnt_hbm,
             shbuf_a, shbuf_b, bxbuf_a, bxbuf_b, labbuf_a, labbuf_b,
             boutbuf_a, boutbuf_b, cntbuf,
             sem_in0, sem_in1, sem_out0, sem_out1):
    cid = lax.axis_index("c")
    sid = lax.axis_index("s")
    wid = cid * _NS + sid
    base_row = wid * _RW

    iot = lax.iota(jnp.int32, 16)
    riot = lax.shift_right_logical(iot, 2)
    neg_inf = jnp.full((16,), -jnp.inf, jnp.float32)
    zero_f = jnp.zeros((16,), jnp.float32)
    zero_i = jnp.zeros((16,), jnp.int32)
    neg1 = jnp.full((16,), -1, jnp.int32)

    sem_in = (sem_in0, sem_in1)
    sem_out = (sem_out0, sem_out1)
    shbufs = (shbuf_a, shbuf_b)
    bxbufs = (bxbuf_a, bxbuf_b)
    labbufs = (labbuf_a, labbuf_b)
    boutbufs = (boutbuf_a, boutbuf_b)
    slab_rows = _NS * _CH
    sc_base = cid * (_R // _NC)

    def start_in(chunk, b):
        slab0 = sc_base + chunk * slab_rows
        row0 = base_row + chunk * _CH

        @pl.when(sid == 0)
        def _():
            pltpu.async_copy(lg_hbm.at[pl.ds(slab0 * _C, slab_rows * _C)],
                             shbufs[b], sem_in[b])
        pltpu.async_copy(bx_hbm.at[pl.ds(row0 * 4, _CH * 4)],
                         bxbufs[b], sem_in[b])

    def wait_in(b):
        @pl.when(sid == 0)
        def _():
            pltpu.make_async_copy(lg_hbm.at[pl.ds(0, slab_rows * _C)],
                                  shbufs[b], sem_in[b]).wait()
        pltpu.make_async_copy(bx_hbm.at[pl.ds(0, _CH * 4)],
                              bxbufs[b], sem_in[b]).wait()
        plsc.subcore_barrier()

    def start_out(chunk, b):
        row0 = base_row + chunk * _CH
        pltpu.async_copy(labbufs[b], lab_hbm.at[pl.ds(row0, _CH)],
                         sem_out[b])
        pltpu.async_copy(boutbufs[b], bout_hbm.at[pl.ds(row0 * 4, _CH * 4)],
                         sem_out[b])

    def wait_out(b):
        pltpu.make_async_copy(labbufs[b], lab_hbm.at[pl.ds(0, _CH)],
                              sem_out[b]).wait()
        pltpu.make_async_copy(boutbufs[b], bout_hbm.at[pl.ds(0, _CH * 4)],
                              sem_out[b]).wait()

    start_in(0, 0)
    start_in(1, 1)

    def pair_body(ci2, acc):
        for b in range(2):
            chunk = ci2 * 2 + b
            labbuf = labbufs[b]
            boutbuf = boutbufs[b]
            bxbuf = bxbufs[b]

            wait_in(b)
            @pl.when(ci2 > 0)
            def _():
                wait_out(b)

            for g in range(_GROUPS):
                labbuf[pl.ds(g * 16, 16)] = neg1
                for i in range(4):
                    boutbuf[pl.ds(g * 64 + i * 16, 16)] = bxbuf[pl.ds(g * 64 + i * 16, 16)]

            start_out(chunk, b)

            @pl.when(chunk + 2 < _NCHUNK)
            def _():
                start_in(chunk + 2, b)
        return acc

    acc = lax.fori_loop(0, _NCHUNK // 2, pair_body,
                        jnp.zeros((16,), jnp.int32))
    wait_out(0)
    wait_out(1)
    cntbuf[...] = acc
    pltpu.sync_copy(cntbuf, cnt_hbm.at[wid])


_sc_call = functools.partial(
    pl.kernel,
    out_type=[
        jax.ShapeDtypeStruct((_R,), jnp.int32),
        jax.ShapeDtypeStruct((_R * 4,), jnp.float32),
        jax.ShapeDtypeStruct((_NW, 16), jnp.int32),
    ],
    mesh=plsc.VectorSubcoreMesh(core_axis_name="c", subcore_axis_name="s"),
    compiler_params=pltpu.CompilerParams(needs_layout_passes=False),
    scratch_types=[
        pltpu.VMEM_SHARED((_NS * _CH * _C,), jnp.float32),
        pltpu.VMEM_SHARED((_NS * _CH * _C,), jnp.float32),
        pltpu.VMEM((_CH * 4,), jnp.float32),
        pltpu.VMEM((_CH * 4,), jnp.float32),
        pltpu.VMEM((_CH,), jnp.int32),
        pltpu.VMEM((_CH,), jnp.int32),
        pltpu.VMEM((_CH * 4,), jnp.float32),
        pltpu.VMEM((_CH * 4,), jnp.float32),
        pltpu.VMEM((16,), jnp.int32),
        pltpu.SemaphoreType.DMA,
        pltpu.SemaphoreType.DMA,
        pltpu.SemaphoreType.DMA,
        pltpu.SemaphoreType.DMA,
    ],
)(_sc_body)


@jax.jit
def kernel(pred_logits, pred_boxes):
    lab, bout, cnt = _sc_call(pred_logits.reshape(_R * _C),
                              pred_boxes.reshape(_R * 4))
    labels = lab.reshape(_B, _Q)
    boxes = bout.reshape(_B, _Q, 4)
    num_boxes = jnp.maximum(cnt[:, 0].sum().astype(jnp.float32), 1.0)
    return labels, boxes, num_boxes


# speedup vs baseline: 1.2135x; 1.2135x over previous
import functools

import jax
import jax.numpy as jnp
from jax import lax
from jax.experimental import pallas as pl
from jax.experimental.pallas import tpu as pltpu

_B, _Q, _C = 64, 2048, 256
_R = _B * _Q
_BR = 2048
_NB = _R // _BR
_GR = _BR // 128


def _tc_body(lg_ref, bx_ref, lab_ref, bout_ref, cnt_ref):
    x0 = lg_ref[:, 0:128]
    x1 = lg_ref[:, 128:256]
    which = x1 > x0
    h = jnp.maximum(x0, x1)
    ii = lax.broadcasted_iota(jnp.int32, (_BR, 128), 1)
    pos = jnp.where(which, ii + 128, ii)
    m = jnp.max(h, axis=1, keepdims=True)
    cand = jnp.where(h >= m, pos, _C)
    a = jnp.min(cand, axis=1, keepdims=True)
    lab_col = jnp.where(m > 0.0, a, -1)
    lab_rep = jnp.broadcast_to(lab_col, (_BR, 128))

    labf = lab_rep.astype(jnp.float32)
    di = lax.broadcasted_iota(jnp.int32, (128, 128), 0)
    dl = lax.broadcasted_iota(jnp.int32, (128, 128), 1)
    eye = (di == dl).astype(jnp.float32)
    ones1 = jnp.ones((1, 128), jnp.float32)
    acc = jnp.zeros((1, 128), jnp.float32)
    diags = []
    for g in range(_GR):
        tile = labf[g * 128:(g + 1) * 128, :] * eye
        d = jax.lax.dot_general(ones1, tile, (((1,), (0,)), ((), ())),
                                preferred_element_type=jnp.float32)
        diags.append(d)
        lab_ref[0, 0, pl.ds(g * 128, 128)] = d[0, :].astype(jnp.int32)
        acc = acc + jnp.where(d >= 0, 1.0, 0.0)
    c = jnp.sum(acc).astype(jnp.int32)
    cnt_ref[...] = jnp.broadcast_to(c, (1, 1, 128))

    v16 = jnp.concatenate(
        [jnp.where(d >= 0, 1.0, 0.0) for d in diags], axis=0)
    ei = lax.broadcasted_iota(jnp.int32, (128, 512), 0)
    el = lax.broadcasted_iota(jnp.int32, (128, 512), 1)
    e = (lax.shift_right_logical(el, 2) == ei).astype(jnp.float32)
    mask = jax.lax.dot_general(v16, e, (((1,), (0,)), ((), ())),
                               preferred_element_type=jnp.float32)
    bout_ref[...] = jnp.where(mask > 0.5, bx_ref[...], 0.0)


def _make_tc_call(interpret=False):
    return pl.pallas_call(
        _tc_body,
        grid=(_NB,),
        in_specs=[
            pl.BlockSpec((_BR, _C), lambda i: (i, 0)),
            pl.BlockSpec((_GR, 512), lambda i: (i, 0)),
        ],
        out_specs=[
            pl.BlockSpec((1, 1, _BR), lambda i: (i, 0, 0)),
            pl.BlockSpec((_GR, 512), lambda i: (i, 0)),
            pl.BlockSpec((1, 1, 128), lambda i: (i, 0, 0)),
        ],
        out_shape=[
            jax.ShapeDtypeStruct((_NB, 1, _BR), jnp.int32),
            jax.ShapeDtypeStruct((_R // 128, 512), jnp.float32),
            jax.ShapeDtypeStruct((_NB, 1, 128), jnp.int32),
        ],
        compiler_params=pltpu.CompilerParams(
            dimension_semantics=("arbitrary",),
        ),
        interpret=interpret,
    )


_tc_call = _make_tc_call()


@jax.jit
def kernel(pred_logits, pred_boxes):
    lab, bout, cnt = _tc_call(pred_logits.reshape(_R, _C),
                              pred_boxes.reshape(_R // 128, 512))
    labels = lab.reshape(_B, _Q)
    boxes = bout.reshape(_B, _Q, 4)
    num_boxes = jnp.maximum(cnt[:, 0, 0].sum().astype(jnp.float32), 1.0)
    return labels, boxes, num_boxes


# speedup vs baseline: 1.3050x; 1.0755x over previous
import functools

import jax
import jax.numpy as jnp
from jax import lax
from jax.experimental import pallas as pl
from jax.experimental.pallas import tpu as pltpu

_B, _Q, _C = 64, 2048, 256
_R = _B * _Q
_BR = 4096
_NB = _R // _BR
_GR = _BR // 128


def _tc_body(lg_ref, bx_ref, lab_ref, bout_ref, cnt_ref):
    x0 = lg_ref[:, 0:128]
    x1 = lg_ref[:, 128:256]
    which = x1 > x0
    h = jnp.maximum(x0, x1)
    ii = lax.broadcasted_iota(jnp.int32, (_BR, 128), 1)
    pos = jnp.where(which, ii + 128, ii)
    m = jnp.max(h, axis=1, keepdims=True)
    cand = jnp.where(h >= m, pos, _C)
    a = jnp.min(cand, axis=1, keepdims=True)
    lab_col = jnp.where(m > 0.0, a, -1)
    lab_rep = jnp.broadcast_to(lab_col, (_BR, 128))

    labf = lab_rep.astype(jnp.float32)
    di = lax.broadcasted_iota(jnp.int32, (128, 128), 0)
    dl = lax.broadcasted_iota(jnp.int32, (128, 128), 1)
    eye = (di == dl).astype(jnp.float32)
    ones1 = jnp.ones((1, 128), jnp.float32)
    dbig = jnp.concatenate(
        [labf[g * 128:(g + 1) * 128, :] * eye for g in range(_GR)],
        axis=1)
    dall = jax.lax.dot_general(ones1, dbig, (((1,), (0,)), ((), ())),
                               preferred_element_type=jnp.float32)
    lab_ref[...] = dall.astype(jnp.int32).reshape(1, 1, _BR)
    vall = jnp.where(dall >= 0, 1.0, 0.0)
    c = jnp.sum(vall).astype(jnp.int32)
    cnt_ref[...] = jnp.broadcast_to(c, (1, 1, 128))

    v16 = vall.reshape(_GR, 128)
    ei = lax.broadcasted_iota(jnp.int32, (128, 512), 0)
    el = lax.broadcasted_iota(jnp.int32, (128, 512), 1)
    e = (lax.shift_right_logical(el, 2) == ei).astype(jnp.float32)
    mask = jax.lax.dot_general(v16, e, (((1,), (0,)), ((), ())),
                               preferred_element_type=jnp.float32)
    bout_ref[...] = jnp.where(mask > 0.5, bx_ref[...], 0.0)


def _make_tc_call(interpret=False):
    return pl.pallas_call(
        _tc_body,
        grid=(_NB,),
        in_specs=[
            pl.BlockSpec((_BR, _C), lambda i: (i, 0)),
            pl.BlockSpec((_GR, 512), lambda i: (i, 0)),
        ],
        out_specs=[
            pl.BlockSpec((1, 1, _BR), lambda i: (i, 0, 0)),
            pl.BlockSpec((_GR, 512), lambda i: (i, 0)),
            pl.BlockSpec((1, 1, 128), lambda i: (i, 0, 0)),
        ],
        out_shape=[
            jax.ShapeDtypeStruct((_NB, 1, _BR), jnp.int32),
            jax.ShapeDtypeStruct((_R // 128, 512), jnp.float32),
            jax.ShapeDtypeStruct((_NB, 1, 128), jnp.int32),
        ],
        compiler_params=pltpu.CompilerParams(
            dimension_semantics=("arbitrary",),
        ),
        interpret=interpret,
    )


_tc_call = _make_tc_call()


@jax.jit
def kernel(pred_logits, pred_boxes):
    lab, bout, cnt = _tc_call(pred_logits.reshape(_R, _C),
                              pred_boxes.reshape(_R // 128, 512))
    labels = lab.reshape(_B, _Q)
    boxes = bout.reshape(_B, _Q, 4)
    num_boxes = jnp.maximum(cnt[:, 0, 0].sum().astype(jnp.float32), 1.0)
    return labels, boxes, num_boxes
